# baseline (speedup 1.0000x reference)
import functools

import jax
import jax.numpy as jnp
from jax import lax
from jax.experimental import pallas as pl
from jax.experimental.pallas import tpu as pltpu
from jax.experimental.pallas import tpu_sc as plsc

HIDDEN = 256
E = 4
NC, NS = 2, 16
NW = NC * NS
CHUNK = 512
BT_ROUTER = 4096
BT_GMM = 512


def _router_kernel(x_ref, wg_ref, idx_ref):
    logits = jnp.dot(x_ref[...], wg_ref[...],
                     preferred_element_type=jnp.float32)
    idx_ref[...] = jnp.argmax(logits, axis=-1).astype(jnp.int32)


def _router(x2, Wg, T):
    return pl.pallas_call(
        _router_kernel,
        grid=(T // BT_ROUTER,),
        in_specs=[
            pl.BlockSpec((BT_ROUTER, HIDDEN), lambda i: (i, 0)),
            pl.BlockSpec((HIDDEN, E), lambda i: (0, 0)),
        ],
        out_specs=pl.BlockSpec((BT_ROUTER,), lambda i: (i,)),
        out_shape=jax.ShapeDtypeStruct((T,), jnp.int32),
    )(x2, Wg)


def _make_sc_sort(T):
    HALF = T // NC
    mesh = plsc.VectorSubcoreMesh(core_axis_name="c", subcore_axis_name="s")

    @functools.partial(
        pl.kernel, mesh=mesh,
        out_type=[
            jax.ShapeDtypeStruct((T, HIDDEN), jnp.float32),
            jax.ShapeDtypeStruct((T,), jnp.int32),
            jax.ShapeDtypeStruct((NC, 16), jnp.int32),
        ],
        scratch_types=[
            pltpu.VMEM((HALF,), jnp.int32),
            pltpu.VMEM((CHUNK // 128, 128), jnp.int32),
            pltpu.VMEM((128, HIDDEN), jnp.float32),
            pltpu.VMEM((16,), jnp.int32),
            pltpu.SemaphoreType.DMA,
        ],
    )
    def k(x_hbm, idx_hbm, xs_hbm, pos_hbm, offs_hbm,
          idx_half, dpos, rows, obuf, sem):
        c = lax.axis_index("c")
        s = lax.axis_index("s")
        half_base = c * HALF
        my_base = half_base + s * CHUNK
        pltpu.sync_copy(idx_hbm.at[pl.ds(half_base, HALF)], idx_half)

        zero16 = jnp.zeros((16,), jnp.int32)
        lane = lax.iota(jnp.int32, 16)

        gdn = lax.GatherDimensionNumbers(
            offset_dims=(), collapsed_slice_dims=(0,), start_index_map=(0,))

        def _shift_down(vv, kk):
            src = jnp.maximum(lane - kk, 0)
            g = lax.gather(vv, src[:, None], gdn, (1,),
                           mode=lax.GatherScatterMode.PROMISE_IN_BOUNDS)
            return jnp.where(lane >= kk, g, zero16)

        def _cumsum16(vv):
            for kk in (1, 2, 4, 8):
                vv = vv + _shift_down(vv, kk)
            return vv

        idx15 = jnp.full((16,), 15, jnp.int32)

        def _bcast_last(vv):
            return lax.gather(vv, idx15[:, None], gdn, (1,),
                              mode=lax.GatherScatterMode.PROMISE_IN_BOUNDS)

        def _popcount(vv):
            return _bcast_last(_cumsum16(vv))

        def body(j, carry):
            t0, t1, t2, t3, p0, p1, p2, p3 = carry
            v = idx_half[pl.ds(j * 16, 16)]
            in_pref = jnp.where(jnp.full((16,), j, jnp.int32) < s * 32,
                                1, 0).astype(jnp.int32)
            ones = [jnp.where(v == e, 1, 0).astype(jnp.int32)
                    for e in range(E)]
            t0 = t0 + ones[0]
            t1 = t1 + ones[1]
            t2 = t2 + ones[2]
            t3 = t3 + ones[3]
            p0 = p0 + in_pref * ones[0]
            p1 = p1 + in_pref * ones[1]
            p2 = p2 + in_pref * ones[2]
            p3 = p3 + in_pref * ones[3]
            return (t0, t1, t2, t3, p0, p1, p2, p3)

        init = (zero16,) * 8
        tots = lax.fori_loop(0, HALF // 16, body, init)
        t = [_popcount(tots[e]) for e in range(E)]
        p = [_popcount(tots[E + e]) for e in range(E)]
        cum = [zero16, t[0], t[0] + t[1], t[0] + t[1] + t[2],
               t[0] + t[1] + t[2] + t[3]]
        hb = jnp.full((16,), half_base, jnp.int32)
        run = [hb + cum[e] + p[e] for e in range(E)]

        @pl.when(s == 0)
        def _():
            offv = zero16
            for e in range(1, E + 1):
                offv = offv + jnp.where(lane == e, cum[e], zero16)
            obuf[...] = offv
            pltpu.sync_copy(obuf, offs_hbm.at[c])

        for ch in range(CHUNK // 128):
            for r in range(8):
                v = idx_half[pl.ds((s * 32 + ch * 8 + r) * 16, 16)]
                dest = zero16
                for e in range(E):
                    m = v == e
                    one = jnp.where(m, 1, 0).astype(jnp.int32)
                    cs = _cumsum16(one)
                    dest = dest + jnp.where(m, run[e] + cs - 1, zero16)
                    run[e] = run[e] + _bcast_last(cs)
                dpos[ch, pl.ds(r * 16, 16)] = dest
            row0 = my_base + ch * 128
            pltpu.sync_copy(x_hbm.at[pl.ds(row0, 128), :], rows)
            pltpu.async_copy(rows, xs_hbm.at[dpos.at[ch]], sem).wait()
            pltpu.sync_copy(dpos.at[ch], pos_hbm.at[pl.ds(row0, 128)])

    return k


def _gmm_kernel(offs_ref, xs_ref, wg_ref, w_ref, b_ref, out_ref):
    i = pl.program_id(0)
    xb = xs_ref[...]
    logits = jnp.dot(xb, wg_ref[...],
                     preferred_element_type=jnp.float32)
    m = jnp.max(logits, axis=-1, keepdims=True)
    ssum = jnp.sum(jnp.exp(logits - m), axis=-1, keepdims=True)
    gate = 1.0 / ssum

    out_ref[...] = jnp.zeros_like(out_ref)
    rowpos = lax.broadcasted_iota(jnp.int32, (xb.shape[0], 1), 0)
    h = i // 16
    lo = (i % 16) * BT_GMM
    for e in range(E):
        seg_lo = offs_ref[h * 16 + e]
        seg_hi = offs_ref[h * 16 + e + 1]

        @pl.when(jnp.logical_and(seg_hi > lo, seg_lo < lo + BT_GMM))
        def _():
            rp = rowpos + lo
            rmask = jnp.logical_and(rp >= seg_lo, rp < seg_hi)
            y = jnp.dot(xb, w_ref[e],
                        preferred_element_type=jnp.float32)
            out_ref[...] += jnp.where(rmask, y + b_ref[e][None, :], 0.0)
    out_ref[...] = out_ref[...] * gate


def _gmm(xs, offs_flat, Wg, W, b, T):
    grid_spec = pltpu.PrefetchScalarGridSpec(
        num_scalar_prefetch=1,
        grid=(T // BT_GMM,),
        in_specs=[
            pl.BlockSpec((BT_GMM, HIDDEN), lambda i, offs: (i, 0)),
            pl.BlockSpec((HIDDEN, E), lambda i, offs: (0, 0)),
            pl.BlockSpec((E, HIDDEN, HIDDEN), lambda i, offs: (0, 0, 0)),
            pl.BlockSpec((E, HIDDEN), lambda i, offs: (0, 0)),
        ],
        out_specs=pl.BlockSpec((BT_GMM, HIDDEN), lambda i, offs: (i, 0)),
    )
    return pl.pallas_call(
        _gmm_kernel,
        grid_spec=grid_spec,
        out_shape=jax.ShapeDtypeStruct((T, HIDDEN), jnp.float32),
    )(offs_flat, xs, Wg, W, b)


def _make_sc_unsort(T):
    mesh = plsc.VectorSubcoreMesh(core_axis_name="c", subcore_axis_name="s")

    @functools.partial(
        pl.kernel, mesh=mesh,
        out_type=jax.ShapeDtypeStruct((T, HIDDEN), jnp.float32),
        scratch_types=[
            pltpu.VMEM((CHUNK // 128, 128), jnp.int32),
            pltpu.VMEM((128, HIDDEN), jnp.float32),
            pltpu.SemaphoreType.DMA,
        ],
    )
    def k(ys_hbm, pos_hbm, out_hbm, posb, rows, sem):
        c = lax.axis_index("c")
        s = lax.axis_index("s")
        my_base = (c * NS + s) * CHUNK
        for ch in range(CHUNK // 128):
            row0 = my_base + ch * 128
            pltpu.sync_copy(pos_hbm.at[pl.ds(row0, 128)], posb.at[ch])
            pltpu.async_copy(ys_hbm.at[posb.at[ch]], rows, sem).wait()
            pltpu.sync_copy(rows, out_hbm.at[pl.ds(row0, 128), :])

    return k


def kernel(x, Wg, W, b):
    orig_shape = x.shape
    x2 = x.reshape(-1, orig_shape[-1])
    T = x2.shape[0]
    idx = _router(x2, Wg, T)
    xs, pos, offs = _make_sc_sort(T)(x2, idx)
    ys = _gmm(xs, offs.reshape(NC * 16), Wg, W, b, T)
    out = _make_sc_unsort(T)(ys, pos)
    return out.reshape(orig_shape)


# speedup vs baseline: 4.4202x; 4.4202x over previous
import jax
import jax.numpy as jnp
from jax.experimental import pallas as pl

HIDDEN = 256
NUM_EXPERTS = 4
BLOCK_T = 8192


def _moe_block_kernel(x_ref, wg_ref, wstack_ref, b_ref, out_ref):
    xb = x_ref[...]
    logits = jnp.dot(xb, wg_ref[...],
                     preferred_element_type=jnp.float32)
    m = jnp.max(logits, axis=-1, keepdims=True)
    s = jnp.sum(jnp.exp(logits - m), axis=-1, keepdims=True)
    gate = 1.0 / s
    idx = jnp.argmax(logits, axis=-1)[:, None]

    sel = [idx == e for e in range(NUM_EXPERTS)]
    xg = gate * xb
    zero = jnp.zeros_like(xg)
    x4 = jnp.concatenate(
        [jnp.where(sel[e], xg, zero) for e in range(NUM_EXPERTS)],
        axis=1)
    wstack = wstack_ref[...].reshape(NUM_EXPERTS * HIDDEN, HIDDEN)
    acc = jnp.dot(x4, wstack,
                  preferred_element_type=jnp.float32)

    bsel = jnp.where(sel[0], b_ref[0][None, :],
           jnp.where(sel[1], b_ref[1][None, :],
           jnp.where(sel[2], b_ref[2][None, :],
                     b_ref[3][None, :])))
    out_ref[...] = acc + gate * bsel


def kernel(x, Wg, W, b):
    orig_shape = x.shape
    x2 = x.reshape(-1, orig_shape[-1])
    T = x2.shape[0]
    grid = (T // BLOCK_T,)
    out = pl.pallas_call(
        _moe_block_kernel,
        grid=grid,
        in_specs=[
            pl.BlockSpec((BLOCK_T, HIDDEN), lambda i: (i, 0)),
            pl.BlockSpec((HIDDEN, NUM_EXPERTS), lambda i: (0, 0)),
            pl.BlockSpec((NUM_EXPERTS, HIDDEN, HIDDEN), lambda i: (0, 0, 0)),
            pl.BlockSpec((NUM_EXPERTS, HIDDEN), lambda i: (0, 0)),
        ],
        out_specs=pl.BlockSpec((BLOCK_T, HIDDEN), lambda i: (i, 0)),
        out_shape=jax.ShapeDtypeStruct((T, HIDDEN), jnp.float32),
    )(x2, Wg, W, b)
    return out.reshape(orig_shape)


# speedup vs baseline: 4.4740x; 1.0122x over previous
import jax
import jax.numpy as jnp
from jax.experimental import pallas as pl

HIDDEN = 256
NUM_EXPERTS = 4
BLOCK_T = 4096


def _moe_block_kernel(x_ref, wg_ref, wstack_ref, b_ref, out_ref):
    xb = x_ref[...]
    logits = jnp.dot(xb, wg_ref[...],
                     preferred_element_type=jnp.float32)
    m = jnp.max(logits, axis=-1, keepdims=True)
    s = jnp.sum(jnp.exp(logits - m), axis=-1, keepdims=True)
    gate = 1.0 / s
    idx = jnp.argmax(logits, axis=-1)[:, None]

    sel = [idx == e for e in range(NUM_EXPERTS)]
    xg = (gate * xb).astype(jnp.bfloat16)
    zero = jnp.zeros_like(xg)
    x4 = jnp.concatenate(
        [jnp.where(sel[e], xg, zero) for e in range(NUM_EXPERTS)],
        axis=1)
    wstack = wstack_ref[...].reshape(NUM_EXPERTS * HIDDEN, HIDDEN)
    wstack = wstack.astype(jnp.bfloat16)
    acc = jnp.dot(x4, wstack,
                  preferred_element_type=jnp.float32)

    bsel = jnp.where(sel[0], b_ref[0][None, :],
           jnp.where(sel[1], b_ref[1][None, :],
           jnp.where(sel[2], b_ref[2][None, :],
                     b_ref[3][None, :])))
    out_ref[...] = acc + gate * bsel


def kernel(x, Wg, W, b):
    orig_shape = x.shape
    x2 = x.reshape(-1, orig_shape[-1])
    T = x2.shape[0]
    grid = (T // BLOCK_T,)
    out = pl.pallas_call(
        _moe_block_kernel,
        grid=grid,
        in_specs=[
            pl.BlockSpec((BLOCK_T, HIDDEN), lambda i: (i, 0)),
            pl.BlockSpec((HIDDEN, NUM_EXPERTS), lambda i: (0, 0)),
            pl.BlockSpec((NUM_EXPERTS, HIDDEN, HIDDEN), lambda i: (0, 0, 0)),
            pl.BlockSpec((NUM_EXPERTS, HIDDEN), lambda i: (0, 0)),
        ],
        out_specs=pl.BlockSpec((BLOCK_T, HIDDEN), lambda i: (i, 0)),
        out_shape=jax.ShapeDtypeStruct((T, HIDDEN), jnp.float32),
    )(x2, Wg, W, b)
    return out.reshape(orig_shape)


# speedup vs baseline: 4.7388x; 1.0592x over previous
import jax
import jax.numpy as jnp
from jax.experimental import pallas as pl
from jax.experimental.pallas import tpu as pltpu

HIDDEN = 256
NUM_EXPERTS = 4
BLOCK_T = 4096


def _moe_block_kernel(x_ref, wg_ref, wstack_ref, b_ref, out_ref):
    xb = x_ref[...]
    logits = jnp.dot(xb, wg_ref[...],
                     preferred_element_type=jnp.float32)
    m = jnp.max(logits, axis=-1, keepdims=True)
    s = jnp.sum(jnp.exp(logits - m), axis=-1, keepdims=True)
    gate = 1.0 / s
    idx = jnp.argmax(logits, axis=-1)[:, None]

    sel = [idx == e for e in range(NUM_EXPERTS)]
    xg = gate * xb
    zero = jnp.zeros_like(xg)
    x4 = jnp.concatenate(
        [jnp.where(sel[e], xg, zero) for e in range(NUM_EXPERTS)],
        axis=1)
    wstack = wstack_ref[...].reshape(NUM_EXPERTS * HIDDEN, HIDDEN)
    acc = jnp.dot(x4, wstack,
                  preferred_element_type=jnp.float32)

    bsel = jnp.where(sel[0], b_ref[0][None, :],
           jnp.where(sel[1], b_ref[1][None, :],
           jnp.where(sel[2], b_ref[2][None, :],
                     b_ref[3][None, :])))
    out_ref[...] = acc + gate * bsel


def kernel(x, Wg, W, b):
    orig_shape = x.shape
    x2 = x.reshape(-1, orig_shape[-1])
    T = x2.shape[0]
    grid = (T // BLOCK_T,)
    out = pl.pallas_call(
        _moe_block_kernel,
        grid=grid,
        compiler_params=pltpu.CompilerParams(
            dimension_semantics=("arbitrary",)),
        in_specs=[
            pl.BlockSpec((BLOCK_T, HIDDEN), lambda i: (i, 0)),
            pl.BlockSpec((HIDDEN, NUM_EXPERTS), lambda i: (0, 0)),
            pl.BlockSpec((NUM_EXPERTS, HIDDEN, HIDDEN), lambda i: (0, 0, 0)),
            pl.BlockSpec((NUM_EXPERTS, HIDDEN), lambda i: (0, 0)),
        ],
        out_specs=pl.BlockSpec((BLOCK_T, HIDDEN), lambda i: (i, 0)),
        out_shape=jax.ShapeDtypeStruct((T, HIDDEN), jnp.float32),
    )(x2, Wg, W, b)
    return out.reshape(orig_shape)
